# baseline (speedup 1.0000x reference)
import numpy as np
import jax
import jax.numpy as jnp
from jax import lax
from jax.experimental import pallas as pl
from jax.experimental.pallas import tpu as pltpu
from jax.experimental.pallas import tpu_sc as plsc

_PAD_ID = 22
_X_ID = 21
_STOP_ID = 20
_B = 4096
_L = 1536
_NWORKERS = 32
_ROWS_PER = _B // _NWORKERS


def _packed_table():
    code = "FFLLSSSSYY**CC*WLLLLPPPPHHQQRRRRIIIMTTTTNNKKSSRRVVVVAAAADDEEGGGG"
    aa_order = "ACDEFGHIKLMNPQRSTVWY"
    aa_to_id = {a: i for i, a in enumerate(aa_order)}
    aa_to_id["*"] = _STOP_ID
    tab = np.full((5, 5, 5), _X_ID, dtype=np.int32)
    idx_map = {0: 2, 1: 1, 2: 3, 3: 0}
    for a in range(4):
        for b in range(4):
            for c in range(4):
                k = idx_map[a] * 16 + idx_map[b] * 4 + idx_map[c]
                tab[a, b, c] = aa_to_id[code[k]]
    rc = np.array([3, 2, 1, 0, 4])
    n = np.arange(5)
    a, b, c, d, e = np.meshgrid(n, n, n, n, n, indexing="ij")
    f0 = tab[a, b, c]
    r0 = tab[rc[c], rc[b], rc[a]]
    f1 = tab[b, c, d]
    r1 = tab[rc[d], rc[c], rc[b]]
    f2 = tab[c, d, e]
    r2 = tab[rc[e], rc[d], rc[c]]
    packed = f0 | (r0 << 5) | (f1 << 10) | (r1 << 15) | (f2 << 20) | (r2 << 25)
    out = np.zeros(3136, dtype=np.int32)
    out[:3125] = packed.reshape(-1)
    return out


_CTAB = _packed_table()


def _sc_body(
    nuc_hbm, ctab_hbm, out_hbm, rb0, rb1, ob0, ob1, ctab_v, si0, si1, so0, so1
):
    wid = lax.axis_index("s") * 2 + lax.axis_index("c")
    pltpu.sync_copy(ctab_hbm, ctab_v)
    lane = lax.iota(jnp.int32, 16)
    iota3 = lane * 3
    pad = jnp.int32(_PAD_ID)
    five_bits = jnp.int32(31)
    base_row = wid * _ROWS_PER
    pad_frame = jnp.where(lane == 0, 4, 5)
    pad_col = jnp.full((16,), 511, jnp.int32)
    for ob in (ob0, ob1):
        plsc.store_scatter(
            ob, [pad_frame, pad_col], jnp.full((16,), pad), mask=lane < 2
        )
    for rb in (rb0, rb1):
        rb[pl.ds(_L, 16)] = jnp.zeros((16,), jnp.int32)

    def in_copy(row, rb, sem):
        return pltpu.make_async_copy(nuc_hbm.at[row], rb.at[pl.ds(0, _L)], sem)

    def out_copy(row, ob, sem):
        return pltpu.make_async_copy(ob, out_hbm.at[row], sem)

    fr0 = jnp.full((16,), 0, jnp.int32)
    fr1 = jnp.full((16,), 1, jnp.int32)
    fr2 = jnp.full((16,), 2, jnp.int32)
    fr3 = jnp.full((16,), 3, jnp.int32)
    fr4 = jnp.full((16,), 4, jnp.int32)
    fr5 = jnp.full((16,), 5, jnp.int32)

    def compute_row(rowbuf, outbuf):
        @plsc.parallel_loop(0, 32, unroll=8)
        def m_body(m):
            posv = m * 48 + iota3
            g0 = plsc.load_gather(rowbuf, [posv])
            g1 = plsc.load_gather(rowbuf, [posv + 1])
            g2 = plsc.load_gather(rowbuf, [posv + 2])
            g3 = plsc.load_gather(rowbuf, [posv + 3])
            g4 = plsc.load_gather(rowbuf, [posv + 4])
            idx5 = (((g0 * 5 + g1) * 5 + g2) * 5 + g3) * 5 + g4
            t = plsc.load_gather(ctab_v, [idx5])
            col = m * 16
            kv = col + lane
            f0 = t & five_bits
            r0 = lax.shift_right_logical(t, 5) & five_bits
            f1 = lax.shift_right_logical(t, 10) & five_bits
            r1 = lax.shift_right_logical(t, 15) & five_bits
            f2 = lax.shift_right_logical(t, 20) & five_bits
            r2 = lax.shift_right_logical(t, 25)
            f1 = jnp.where(kv == 511, pad, f1)
            f2 = jnp.where(kv == 511, pad, f2)
            plsc.store_scatter(outbuf, [fr0, kv], f0)
            plsc.store_scatter(outbuf, [fr1, kv], f1)
            plsc.store_scatter(outbuf, [fr2, kv], f2)
            plsc.store_scatter(outbuf, [fr3, 511 - kv], r0)
            plsc.store_scatter(outbuf, [fr5, 510 - kv], r1, mask=kv <= 510)
            plsc.store_scatter(outbuf, [fr4, 510 - kv], r2, mask=kv <= 510)

    in_copy(base_row, rb0, si0).start()
    in_copy(base_row + 1, rb1, si1).start()

    def pair_body(i, carry):
        for s, (rb, ob, si, so) in enumerate(
            ((rb0, ob0, si0, so0), (rb1, ob1, si1, so1))
        ):
            row = base_row + 2 * i + s
            in_copy(row, rb, si).wait()

            @pl.when(i > 0)
            def _():
                out_copy(row, ob, so).wait()

            compute_row(rb, ob)
            nxt = jnp.minimum(row + 2, _B - 1)
            in_copy(nxt, rb, si).start()
            out_copy(row, ob, so).start()
        return carry

    lax.fori_loop(0, _ROWS_PER // 2, pair_body, 0)
    last = base_row + _ROWS_PER - 2
    out_copy(last, ob0, so0).wait()
    out_copy(last + 1, ob1, so1).wait()
    in_copy(last, rb0, si0).wait()
    in_copy(last + 1, rb1, si1).wait()


def kernel(nucleotide_ids):
    ctab = jnp.asarray(_CTAB)
    mesh = plsc.VectorSubcoreMesh(core_axis_name="c", subcore_axis_name="s")
    run = pl.kernel(
        _sc_body,
        out_type=jax.ShapeDtypeStruct((_B, 6, 512), jnp.int32),
        mesh=mesh,
        scratch_types=[
            pltpu.VMEM((_L + 16,), jnp.int32),
            pltpu.VMEM((_L + 16,), jnp.int32),
            pltpu.VMEM((6, 512), jnp.int32),
            pltpu.VMEM((6, 512), jnp.int32),
            pltpu.VMEM((3136,), jnp.int32),
            pltpu.SemaphoreType.DMA,
            pltpu.SemaphoreType.DMA,
            pltpu.SemaphoreType.DMA,
            pltpu.SemaphoreType.DMA,
        ],
        compiler_params=pltpu.CompilerParams(needs_layout_passes=False),
    )
    aa_ids = run(nucleotide_ids, ctab)
    frame_lengths = jnp.broadcast_to(
        jnp.asarray([512, 511, 511, 512, 511, 511], jnp.int32), (_B, 6)
    )
    return aa_ids, frame_lengths


# speedup vs baseline: 1.0269x; 1.0269x over previous
import numpy as np
import jax
import jax.numpy as jnp
from jax import lax
from jax.experimental import pallas as pl
from jax.experimental.pallas import tpu as pltpu
from jax.experimental.pallas import tpu_sc as plsc

_PAD_ID = 22
_X_ID = 21
_STOP_ID = 20
_B = 4096
_L = 1536
_NWORKERS = 32
_ROWS_PER = _B // _NWORKERS


def _packed_table():
    code = "FFLLSSSSYY**CC*WLLLLPPPPHHQQRRRRIIIMTTTTNNKKSSRRVVVVAAAADDEEGGGG"
    aa_order = "ACDEFGHIKLMNPQRSTVWY"
    aa_to_id = {a: i for i, a in enumerate(aa_order)}
    aa_to_id["*"] = _STOP_ID
    tab = np.full((5, 5, 5), _X_ID, dtype=np.int32)
    idx_map = {0: 2, 1: 1, 2: 3, 3: 0}
    for a in range(4):
        for b in range(4):
            for c in range(4):
                k = idx_map[a] * 16 + idx_map[b] * 4 + idx_map[c]
                tab[a, b, c] = aa_to_id[code[k]]
    rc = np.array([3, 2, 1, 0, 4])
    n = np.arange(5)
    a, b, c, d, e = np.meshgrid(n, n, n, n, n, indexing="ij")
    f0 = tab[a, b, c]
    r0 = tab[rc[c], rc[b], rc[a]]
    f1 = tab[b, c, d]
    r1 = tab[rc[d], rc[c], rc[b]]
    f2 = tab[c, d, e]
    r2 = tab[rc[e], rc[d], rc[c]]
    packed = f0 | (r0 << 5) | (f1 << 10) | (r1 << 15) | (f2 << 20) | (r2 << 25)
    out = np.zeros(3136, dtype=np.int32)
    out[:3125] = packed.reshape(-1)
    return out


_CTAB = _packed_table()


def _sc_body(
    nuc_hbm, ctab_hbm, out_hbm, rb0, rb1, ob0, ob1, ctab_v, si0, si1, so0, so1
):
    wid = lax.axis_index("s") * 2 + lax.axis_index("c")
    pltpu.sync_copy(ctab_hbm, ctab_v)
    lane = lax.iota(jnp.int32, 16)
    iota3 = lane * 3
    pad = jnp.int32(_PAD_ID)
    five_bits = jnp.int32(31)
    base_row = wid * _ROWS_PER
    pad_frame = jnp.where(lane == 0, 4, 5)
    pad_col = jnp.full((16,), 511, jnp.int32)
    for ob in (ob0, ob1):
        plsc.store_scatter(
            ob, [pad_frame, pad_col], jnp.full((16,), pad), mask=lane < 2
        )
    for rb in (rb0, rb1):
        rb[pl.ds(_L, 16)] = jnp.zeros((16,), jnp.int32)

    def in_copy(row, rb, sem):
        return pltpu.make_async_copy(nuc_hbm.at[row], rb.at[pl.ds(0, _L)], sem)

    def out_copy(row, ob, sem):
        return pltpu.make_async_copy(ob, out_hbm.at[row], sem)

    fr0 = jnp.full((16,), 0, jnp.int32)
    fr1 = jnp.full((16,), 1, jnp.int32)
    fr2 = jnp.full((16,), 2, jnp.int32)
    fr3 = jnp.full((16,), 3, jnp.int32)
    fr4 = jnp.full((16,), 4, jnp.int32)
    fr5 = jnp.full((16,), 5, jnp.int32)

    def compute_row(rowbuf, outbuf):
        @plsc.parallel_loop(0, 32, unroll=4)
        def m_body(m):
            posv = m * 48 + iota3
            g0 = plsc.load_gather(rowbuf, [posv])
            g1 = plsc.load_gather(rowbuf, [posv + 1])
            g2 = plsc.load_gather(rowbuf, [posv + 2])
            g3 = plsc.load_gather(rowbuf, [posv + 3])
            g4 = plsc.load_gather(rowbuf, [posv + 4])
            idx5 = (g0 * 25 + g1 * 5 + g2) * 25 + (g3 * 5 + g4)
            t = plsc.load_gather(ctab_v, [idx5])
            col = m * 16
            kv = col + lane
            f0 = t & five_bits
            r0 = lax.shift_right_logical(t, 5) & five_bits
            f1 = lax.shift_right_logical(t, 10) & five_bits
            r1 = lax.shift_right_logical(t, 15) & five_bits
            f2 = lax.shift_right_logical(t, 20) & five_bits
            r2 = lax.shift_right_logical(t, 25)
            plsc.store_scatter(outbuf, [fr0, kv], f0)
            plsc.store_scatter(outbuf, [fr1, kv], f1)
            plsc.store_scatter(outbuf, [fr2, kv], f2)
            edge = kv <= 510
            plsc.store_scatter(outbuf, [fr3, 511 - kv], r0)
            plsc.store_scatter(outbuf, [fr5, 510 - kv], r1, mask=edge)
            plsc.store_scatter(outbuf, [fr4, 510 - kv], r2, mask=edge)

        fwd_pad_frame = jnp.where(lane == 0, 1, 2)
        plsc.store_scatter(
            outbuf,
            [fwd_pad_frame, jnp.full((16,), 511, jnp.int32)],
            jnp.full((16,), pad),
            mask=lane < 2,
        )

    in_copy(base_row, rb0, si0).start()
    in_copy(base_row + 1, rb1, si1).start()

    def pair_body(i, carry):
        for s, (rb, ob, si, so) in enumerate(
            ((rb0, ob0, si0, so0), (rb1, ob1, si1, so1))
        ):
            row = base_row + 2 * i + s
            in_copy(row, rb, si).wait()

            @pl.when(i > 0)
            def _():
                out_copy(row, ob, so).wait()

            compute_row(rb, ob)
            nxt = jnp.minimum(row + 2, _B - 1)
            in_copy(nxt, rb, si).start()
            out_copy(row, ob, so).start()
        return carry

    lax.fori_loop(0, _ROWS_PER // 2, pair_body, 0)
    last = base_row + _ROWS_PER - 2
    out_copy(last, ob0, so0).wait()
    out_copy(last + 1, ob1, so1).wait()
    in_copy(last, rb0, si0).wait()
    in_copy(last + 1, rb1, si1).wait()


def kernel(nucleotide_ids):
    ctab = jnp.asarray(_CTAB)
    mesh = plsc.VectorSubcoreMesh(core_axis_name="c", subcore_axis_name="s")
    run = pl.kernel(
        _sc_body,
        out_type=jax.ShapeDtypeStruct((_B, 6, 512), jnp.int32),
        mesh=mesh,
        scratch_types=[
            pltpu.VMEM((_L + 16,), jnp.int32),
            pltpu.VMEM((_L + 16,), jnp.int32),
            pltpu.VMEM((6, 512), jnp.int32),
            pltpu.VMEM((6, 512), jnp.int32),
            pltpu.VMEM((3136,), jnp.int32),
            pltpu.SemaphoreType.DMA,
            pltpu.SemaphoreType.DMA,
            pltpu.SemaphoreType.DMA,
            pltpu.SemaphoreType.DMA,
        ],
        compiler_params=pltpu.CompilerParams(needs_layout_passes=False),
    )
    aa_ids = run(nucleotide_ids, ctab)
    frame_lengths = jnp.broadcast_to(
        jnp.asarray([512, 511, 511, 512, 511, 511], jnp.int32), (_B, 6)
    )
    return aa_ids, frame_lengths
